# baseline (speedup 1.0000x reference)
import functools

import jax
import jax.numpy as jnp
import numpy as np
from jax import lax
from jax.experimental import pallas as pl

BBOX_XFORM_CLIP = float(np.log(1000.0 / 16.0))
PRE_NMS_TOP_K = 1000
NUM_PROPOSALS = 1000
NMS_IOU = 0.7


def _sigmoid_body(x_ref, o_ref):
    o_ref[...] = jax.nn.sigmoid(x_ref[...])


def _sigmoid_pallas(x):
    return pl.pallas_call(
        _sigmoid_body,
        out_shape=jax.ShapeDtypeStruct(x.shape, x.dtype),
    )(x)


def _decode_boxes(boxes, anchors):
    ay1 = anchors[..., 0]; ax1 = anchors[..., 1]; ay2 = anchors[..., 2]; ax2 = anchors[..., 3]
    ah = ay2 - ay1; aw = ax2 - ax1
    ayc = ay1 + 0.5 * ah; axc = ax1 + 0.5 * aw
    dy = boxes[..., 0]; dx = boxes[..., 1]
    dh = jnp.minimum(boxes[..., 2], BBOX_XFORM_CLIP)
    dw = jnp.minimum(boxes[..., 3], BBOX_XFORM_CLIP)
    nyc = dy * ah + ayc; nxc = dx * aw + axc
    nh = jnp.exp(dh) * ah; nw = jnp.exp(dw) * aw
    return jnp.stack([nyc - 0.5 * nh, nxc - 0.5 * nw, nyc + 0.5 * nh, nxc + 0.5 * nw], axis=-1)


def _clip_boxes(boxes, image_shape):
    h = image_shape[..., 0:1]; w = image_shape[..., 1:2]
    maxes = jnp.concatenate([h, w, h, w], axis=-1)
    return jnp.clip(boxes, 0.0, maxes)


def _top_k_boxes(boxes, scores, k):
    sc, idx = lax.top_k(scores, k)
    bx = jnp.take_along_axis(boxes, idx[..., None], axis=1)
    return bx, sc


def _self_iou(boxes):
    y1 = boxes[..., 0]; x1 = boxes[..., 1]; y2 = boxes[..., 2]; x2 = boxes[..., 3]
    area = jnp.maximum(y2 - y1, 0.0) * jnp.maximum(x2 - x1, 0.0)
    iy1 = jnp.maximum(y1[:, :, None], y1[:, None, :])
    ix1 = jnp.maximum(x1[:, :, None], x1[:, None, :])
    iy2 = jnp.minimum(y2[:, :, None], y2[:, None, :])
    ix2 = jnp.minimum(x2[:, :, None], x2[:, None, :])
    inter = jnp.maximum(iy2 - iy1, 0.0) * jnp.maximum(ix2 - ix1, 0.0)
    union = area[:, :, None] + area[:, None, :] - inter
    return inter / (union + 1e-8)


def _sorted_nms_padded(scores, boxes, max_output_size, iou_threshold):
    B, K = scores.shape
    iou = _self_iou(boxes)
    idxs = jnp.arange(K)

    def body(keep, i):
        keep_i = keep[:, i]
        suppress = (iou[:, i, :] > iou_threshold) & keep_i[:, None] & (idxs[None, :] > i)
        return keep & ~suppress, None

    keep, _ = lax.scan(body, jnp.ones((B, K), dtype=bool), idxs)
    masked = jnp.where(keep, scores, -1.0)
    sel_masked, idx = lax.top_k(masked, max_output_size)
    valid = sel_masked > -0.5
    sel_sc = jnp.where(valid, jnp.take_along_axis(scores, idx, axis=1), 0.0)
    sel_bx = jnp.where(valid[..., None], jnp.take_along_axis(boxes, idx[..., None], axis=1), 0.0)
    return sel_sc, sel_bx


def kernel(raw_boxes_l3, raw_scores_l3, anchor_boxes_l3,
           raw_boxes_l4, raw_scores_l4, anchor_boxes_l4,
           raw_boxes_l5, raw_scores_l5, anchor_boxes_l5,
           image_shape):
    levels = [
        (raw_boxes_l3, raw_scores_l3, anchor_boxes_l3),
        (raw_boxes_l4, raw_scores_l4, anchor_boxes_l4),
        (raw_boxes_l5, raw_scores_l5, anchor_boxes_l5),
    ]
    img = image_shape[:, None, :]
    rois_list, score_list = [], []
    for rb, rs, ab in levels:
        B, fh, fw, na = rs.shape
        nb = fh * fw * na
        sc = _sigmoid_pallas(jnp.reshape(rs, (B, nb)))
        bx = jnp.reshape(rb, (B, nb, 4))
        an = jnp.reshape(ab, (B, nb, 4)).astype(sc.dtype)
        bx = _decode_boxes(bx, an)
        bx = _clip_boxes(bx, img)
        pre_k = min(nb, PRE_NMS_TOP_K)
        post_k = min(nb, NUM_PROPOSALS)
        bx, sc = _top_k_boxes(bx, sc, pre_k)
        sc, bx = _sorted_nms_padded(sc, bx, post_k, NMS_IOU)
        rois_list.append(bx)
        score_list.append(sc)
    all_rois = jnp.concatenate(rois_list, axis=1)
    all_scores = jnp.concatenate(score_list, axis=1)
    k = min(all_scores.shape[1], NUM_PROPOSALS)
    selected_rois, selected_scores = _top_k_boxes(all_rois, all_scores, k)
    return (selected_rois, selected_scores)


# speedup vs baseline: 7.7205x; 7.7205x over previous
import functools

import jax
import jax.numpy as jnp
import numpy as np
from jax import lax
from jax.experimental import pallas as pl
from jax.experimental.pallas import tpu as pltpu

BBOX_XFORM_CLIP = float(np.log(1000.0 / 16.0))
PRE_NMS_TOP_K = 1000
NUM_PROPOSALS = 1000
NMS_IOU = 0.7

_LEVEL_N2 = (16384, 4096, 1024)
_LEVEL_NB = (12288, 3072, 768)


def _iotas(R):
    lane = lax.broadcasted_iota(jnp.int32, (1, 128), 1)
    row = lax.broadcasted_iota(jnp.int32, (R, 1), 0)
    return lane, row


def _bitonic_sort_desc(key, idx, vals, R):
    n = R * 128
    lane, row = _iotas(R)
    k = 2
    while k <= n:
        j = k // 2
        while j >= 1:
            if j < 128:
                m_lo = (lane & j) == 0
                axis, jr, sz = 1, j, 128
            else:
                m_lo = (row & (j // 128)) == 0
                axis, jr, sz = 0, j // 128, R

            def prt(v, axis=axis, jr=jr, sz=sz, m_lo=m_lo):
                return jnp.where(m_lo, pltpu.roll(v, sz - jr, axis),
                                 pltpu.roll(v, jr, axis))

            pk = prt(key)
            pidx = prt(idx)
            pvals = [prt(v) for v in vals]
            if k < 128:
                d = (lane & k) == 0
            else:
                d = (row & (k // 128)) == 0
            keep_max = m_lo == d
            eq = pk == key
            gt = (pk > key) | (eq & (pidx < idx))
            lt = (pk < key) | (eq & (pidx > idx))
            take = (keep_max & gt) | (~keep_max & lt)
            key = jnp.where(take, pk, key)
            idx = jnp.where(take, pidx, idx)
            vals = [jnp.where(take, pv, v) for pv, v in zip(pvals, vals)]
            j //= 2
        k *= 2
    return key, idx, vals


def _col_bcast(rowvec):
    return jnp.transpose(jnp.broadcast_to(rowvec, (128, 128)), (1, 0))


def _decode_clip(bx, an, h, w):
    ay1, ax1, ay2, ax2 = an
    dy, dx, dh, dw = bx
    ah = ay2 - ay1
    aw = ax2 - ax1
    ayc = ay1 + 0.5 * ah
    axc = ax1 + 0.5 * aw
    dh = jnp.minimum(dh, BBOX_XFORM_CLIP)
    dw = jnp.minimum(dw, BBOX_XFORM_CLIP)
    nyc = dy * ah + ayc
    nxc = dx * aw + axc
    nh = jnp.exp(dh) * ah
    nw = jnp.exp(dw) * aw
    y1 = jnp.clip(nyc - 0.5 * nh, 0.0, h)
    x1 = jnp.clip(nxc - 0.5 * nw, 0.0, w)
    y2 = jnp.clip(nyc + 0.5 * nh, 0.0, h)
    x2 = jnp.clip(nxc + 0.5 * nw, 0.0, w)
    return [y1, x1, y2, x2]


def _nms_1024(key8, c8, pre_k, iou_scr, keptT_scr):
    lane128, _ = _iotas(8)
    lane_f = lane128
    y1, x1, y2, x2 = c8
    for rp in range(8):
        y1T = _col_bcast(y1[rp:rp + 1])
        x1T = _col_bcast(x1[rp:rp + 1])
        y2T = _col_bcast(y2[rp:rp + 1])
        x2T = _col_bcast(x2[rp:rp + 1])
        areaT = jnp.maximum(y2T - y1T, 0.0) * jnp.maximum(x2T - x1T, 0.0)
        for r in range(rp, 8):
            y1B = jnp.broadcast_to(y1[r:r + 1], (128, 128))
            x1B = jnp.broadcast_to(x1[r:r + 1], (128, 128))
            y2B = jnp.broadcast_to(y2[r:r + 1], (128, 128))
            x2B = jnp.broadcast_to(x2[r:r + 1], (128, 128))
            areaB = jnp.maximum(y2B - y1B, 0.0) * jnp.maximum(x2B - x1B, 0.0)
            iy = jnp.maximum(jnp.minimum(y2T, y2B) - jnp.maximum(y1T, y1B), 0.0)
            ix = jnp.maximum(jnp.minimum(x2T, x2B) - jnp.maximum(x1T, x1B), 0.0)
            inter = iy * ix
            iou = inter / (areaT + areaB - inter + 1e-8)
            iou_scr[rp * 128:(rp + 1) * 128, r * 128:(r + 1) * 128] = iou

    pos = lax.broadcasted_iota(jnp.int32, (8, 128), 0) * 128 + \
        lax.broadcasted_iota(jnp.int32, (8, 128), 1)
    valid = (pos < pre_k).astype(jnp.float32)

    kept_rows = []
    for r in range(8):
        act = valid[r:r + 1]
        if r > 0:
            vals = iou_scr[0:r * 128, r * 128:(r + 1) * 128]
            kt = keptT_scr[0:r * 128, :]
            supp = jnp.max(jnp.where(vals > NMS_IOU, kt, 0.0), axis=0, keepdims=True)
            act = jnp.where(supp > 0.5, 0.0, act)

        subl = lax.broadcasted_iota(jnp.int32, (8, 128), 0)

        def body(jj, a, r=r):
            base = pl.multiple_of((jj // 8) * 8, 8)
            blk = iou_scr[pl.ds(r * 128 + base, 8), pl.ds(r * 128, 128)]
            iou_row = jnp.max(jnp.where(subl == (jj - base), blk, -1.0),
                              axis=0, keepdims=True)
            aj = jnp.sum(jnp.where(lane_f == jj, a, 0.0))
            supp = (iou_row > NMS_IOU) & (lane_f > jj) & (aj > 0.5)
            return jnp.where(supp, 0.0, a)

        act = lax.fori_loop(0, 128, body, act)
        kept_rows.append(act)
        if r < 7:
            keptT_scr[r * 128:(r + 1) * 128, :] = _col_bcast(act)
    return jnp.concatenate(kept_rows, axis=0)


def _roi_kernel(sc3, bx3, an3, sc4, bx4, an4, sc5, bx5, an5, img,
                sc_out, bx_out, iou_scr, keptT_scr):
    b = pl.program_id(0)
    h = img[b, 0]
    w = img[b, 1]

    level_scores = []
    level_coords = []
    for (sc_ref, bx_ref, an_ref, n2, nb) in (
            (sc3, bx3, an3, 16384, 12288),
            (sc4, bx4, an4, 4096, 3072),
            (sc5, bx5, an5, 1024, 768)):
        R = n2 // 128
        pre_k = min(nb, PRE_NMS_TOP_K)
        key = jax.nn.sigmoid(sc_ref[0])
        pos = lax.broadcasted_iota(jnp.int32, (R, 128), 0) * 128 + \
            lax.broadcasted_iota(jnp.int32, (R, 128), 1)
        bx = [bx_ref[0, i] for i in range(4)]
        an = [an_ref[0, i] for i in range(4)]
        coords = _decode_clip(bx, an, h, w)
        key_s, idx_s, coords_s = _bitonic_sort_desc(key, pos, coords, R)
        k8 = key_s[0:8]
        idx8 = idx_s[0:8]
        c8 = [c[0:8] for c in coords_s]
        kept = _nms_1024(k8, c8, pre_k, iou_scr, keptT_scr)
        masked = jnp.where(kept > 0.5, k8, -1.0)
        mkey, _, mc = _bitonic_sort_desc(masked, idx8, c8, 8)
        msc = jnp.maximum(mkey, 0.0)
        mc = [jnp.where(mkey > -0.5, c, 0.0) for c in mc]
        level_scores.append(msc)
        level_coords.append(mc)

    pad = jnp.full((8, 128), -1.0, dtype=jnp.float32)
    allsc = jnp.concatenate(level_scores + [pad], axis=0)
    zpad = jnp.zeros((8, 128), dtype=jnp.float32)
    allc = [jnp.concatenate([lc[i] for lc in level_coords] + [zpad], axis=0)
            for i in range(4)]
    mpos = lax.broadcasted_iota(jnp.int32, (32, 128), 0) * 128 + \
        lax.broadcasted_iota(jnp.int32, (32, 128), 1)
    fkey, _, fc = _bitonic_sort_desc(allsc, mpos, allc, 32)
    sc_out[0] = fkey[0:8]
    for i in range(4):
        bx_out[0, i] = fc[i][0:8]


def _prep_level(rs, rb, ab, n2):
    B = rs.shape[0]
    nb = rs.shape[1] * rs.shape[2] * rs.shape[3]
    R = n2 // 128
    sc = jnp.reshape(rs, (B, nb))
    sc = jnp.pad(sc, ((0, 0), (0, n2 - nb)), constant_values=-jnp.inf)
    sc = jnp.reshape(sc, (B, R, 128))
    bx = jnp.transpose(jnp.reshape(rb, (B, nb, 4)), (0, 2, 1))
    bx = jnp.reshape(jnp.pad(bx, ((0, 0), (0, 0), (0, n2 - nb))), (B, 4, R, 128))
    an = jnp.transpose(jnp.reshape(ab, (B, nb, 4)), (0, 2, 1))
    an = jnp.reshape(jnp.pad(an, ((0, 0), (0, 0), (0, n2 - nb))), (B, 4, R, 128))
    return sc, bx, an


@functools.partial(jax.jit, static_argnames=("interpret",))
def _run(raw_boxes_l3, raw_scores_l3, anchor_boxes_l3,
         raw_boxes_l4, raw_scores_l4, anchor_boxes_l4,
         raw_boxes_l5, raw_scores_l5, anchor_boxes_l5,
         image_shape, interpret=False):
    B = raw_scores_l3.shape[0]
    sc3, bx3, an3 = _prep_level(raw_scores_l3, raw_boxes_l3, anchor_boxes_l3, 16384)
    sc4, bx4, an4 = _prep_level(raw_scores_l4, raw_boxes_l4, anchor_boxes_l4, 4096)
    sc5, bx5, an5 = _prep_level(raw_scores_l5, raw_boxes_l5, anchor_boxes_l5, 1024)

    def bs(shape):
        return pl.BlockSpec(shape, lambda b: (b,) + (0,) * (len(shape) - 1))

    sc_out, bx_out = pl.pallas_call(
        _roi_kernel,
        grid=(B,),
        in_specs=[
            bs((1, 128, 128)), bs((1, 4, 128, 128)), bs((1, 4, 128, 128)),
            bs((1, 32, 128)), bs((1, 4, 32, 128)), bs((1, 4, 32, 128)),
            bs((1, 8, 128)), bs((1, 4, 8, 128)), bs((1, 4, 8, 128)),
            pl.BlockSpec(memory_space=pltpu.SMEM),
        ],
        out_specs=[bs((1, 8, 128)), bs((1, 4, 8, 128))],
        out_shape=[
            jax.ShapeDtypeStruct((B, 8, 128), jnp.float32),
            jax.ShapeDtypeStruct((B, 4, 8, 128), jnp.float32),
        ],
        scratch_shapes=[
            pltpu.VMEM((1024, 1024), jnp.float32),
            pltpu.VMEM((1024, 128), jnp.float32),
        ],
        interpret=interpret,
    )(sc3, bx3, an3, sc4, bx4, an4, sc5, bx5, an5, image_shape)

    scores = jnp.reshape(sc_out, (B, 1024))[:, :NUM_PROPOSALS]
    rois = jnp.transpose(jnp.reshape(bx_out, (B, 4, 1024)), (0, 2, 1))[:, :NUM_PROPOSALS, :]
    return rois, scores


def kernel(raw_boxes_l3, raw_scores_l3, anchor_boxes_l3,
           raw_boxes_l4, raw_scores_l4, anchor_boxes_l4,
           raw_boxes_l5, raw_scores_l5, anchor_boxes_l5,
           image_shape):
    return _run(raw_boxes_l3, raw_scores_l3, anchor_boxes_l3,
                raw_boxes_l4, raw_scores_l4, anchor_boxes_l4,
                raw_boxes_l5, raw_scores_l5, anchor_boxes_l5,
                image_shape)


# speedup vs baseline: 41.4241x; 5.3655x over previous
import functools

import jax
import jax.numpy as jnp
import numpy as np
from jax import lax
from jax.experimental import pallas as pl
from jax.experimental.pallas import tpu as pltpu

BBOX_XFORM_CLIP = float(np.log(1000.0 / 16.0))
PRE_NMS_TOP_K = 1000
NUM_PROPOSALS = 1000
NMS_IOU = 0.7

_B = 4
_NLVL = 3
_NP = _B * _NLVL
_LVL_N2 = (16384, 4096, 1024)
_LVL_NB = (12288, 3072, 768)
_LVL_PREK = tuple(min(nb, PRE_NMS_TOP_K) for nb in _LVL_NB)


def _lane_iota(R=1):
    return lax.broadcasted_iota(jnp.int32, (R, 128), 1)[0:1]


def _bitonic_sort_desc(key, idx, vals, R_local):
    R = key.shape[0]
    n = R_local * 128
    lane = lax.broadcasted_iota(jnp.int32, (R, 128), 1)[0:1]
    row = lax.broadcasted_iota(jnp.int32, (R, 1), 0)
    rowl = row & (R_local - 1)
    k = 2
    while k <= n:
        j = k // 2
        while j >= 1:
            if j < 128:
                m_lo = (lane & j) == 0
                axis, jr, sz = 1, j, 128
            else:
                m_lo = (rowl & (j // 128)) == 0
                axis, jr, sz = 0, j // 128, R

            def prt(v, axis=axis, jr=jr, sz=sz, m_lo=m_lo):
                return jnp.where(m_lo, pltpu.roll(v, sz - jr, axis),
                                 pltpu.roll(v, jr, axis))

            pk = prt(key)
            pidx = prt(idx)
            pvals = [prt(v) for v in vals]
            if k < 128:
                d = (lane & k) == 0
            else:
                d = (rowl & (k // 128)) == 0
            keep_max = m_lo == d
            eq = pk == key
            gt = (pk > key) | (eq & (pidx < idx))
            lt = (pk < key) | (eq & (pidx > idx))
            take = (keep_max & gt) | (~keep_max & lt)
            key = jnp.where(take, pk, key)
            idx = jnp.where(take, pidx, idx)
            vals = [jnp.where(take, pv, v) for pv, v in zip(pvals, vals)]
            j //= 2
        k *= 2
    return key, idx, vals


def _col_bcast(rowvec):
    return jnp.transpose(jnp.broadcast_to(rowvec, (128, 128)), (1, 0))


def _col_to_row(col):
    return jnp.transpose(jnp.broadcast_to(col, (128, 128)), (1, 0))[0:1]


def _decode_clip(bx, an, hcol, wcol):
    ay1, ax1, ay2, ax2 = an
    dy, dx, dh, dw = bx
    ah = ay2 - ay1
    aw = ax2 - ax1
    ayc = ay1 + 0.5 * ah
    axc = ax1 + 0.5 * aw
    dh = jnp.minimum(dh, BBOX_XFORM_CLIP)
    dw = jnp.minimum(dw, BBOX_XFORM_CLIP)
    nyc = dy * ah + ayc
    nxc = dx * aw + axc
    nh = jnp.exp(dh) * ah
    nw = jnp.exp(dw) * aw
    zero = jnp.float32(0.0)
    y1 = jnp.clip(nyc - 0.5 * nh, zero, hcol)
    x1 = jnp.clip(nxc - 0.5 * nw, zero, wcol)
    y2 = jnp.clip(nyc + 0.5 * nh, zero, hcol)
    x2 = jnp.clip(nxc + 0.5 * nw, zero, wcol)
    return [y1, x1, y2, x2]


def _iou_strip(cT, areaT, crow):
    y1T, x1T, y2T, x2T = cT
    y1B = jnp.broadcast_to(crow[0], (128, 128))
    x1B = jnp.broadcast_to(crow[1], (128, 128))
    y2B = jnp.broadcast_to(crow[2], (128, 128))
    x2B = jnp.broadcast_to(crow[3], (128, 128))
    areaB = jnp.maximum(y2B - y1B, 0.0) * jnp.maximum(x2B - x1B, 0.0)
    iy = jnp.maximum(jnp.minimum(y2T, y2B) - jnp.maximum(y1T, y1B), 0.0)
    ix = jnp.maximum(jnp.minimum(x2T, x2B) - jnp.maximum(x1T, x1B), 0.0)
    inter = iy * ix
    return inter / (areaT + areaB - inter + 1e-8)


def _roi_kernel(sc3, bx3, an3, sc4, bx4, an4, sc5, bx5, an5, img,
                sc_out, bx_out, iou_scr):
    lane = _lane_iota()

    k8 = [None] * _NP
    idx8 = [None] * _NP
    c8 = [None] * _NP
    for l, (sc_ref, bx_ref, an_ref) in enumerate(
            ((sc3, bx3, an3), (sc4, bx4, an4), (sc5, bx5, an5))):
        R = _LVL_N2[l] // 128
        key = jax.nn.sigmoid(
            jnp.concatenate([sc_ref[b] for b in range(_B)], axis=0))
        RT = _B * R
        row = lax.broadcasted_iota(jnp.int32, (RT, 1), 0)
        pos = (row & (R - 1)) * 128 + \
            lax.broadcasted_iota(jnp.int32, (RT, 128), 1)
        coords_raw = [jnp.concatenate([bx_ref[b, i] for b in range(_B)], axis=0)
                      for i in range(4)]
        anchors = [jnp.concatenate([an_ref[b, i] for b in range(_B)], axis=0)
                   for i in range(4)]
        hcol = jnp.concatenate(
            [jnp.zeros((R, 1), jnp.float32) + img[b, 0] for b in range(_B)], axis=0)
        wcol = jnp.concatenate(
            [jnp.zeros((R, 1), jnp.float32) + img[b, 1] for b in range(_B)], axis=0)
        coords = _decode_clip(coords_raw, anchors, hcol, wcol)
        key_s, idx_s, coords_s = _bitonic_sort_desc(key, pos, coords, R)
        for b in range(_B):
            p = b * _NLVL + l
            k8[p] = key_s[b * R:b * R + 8]
            idx8[p] = idx_s[b * R:b * R + 8]
            c8[p] = [c[b * R:b * R + 8] for c in coords_s]

    prek = [_LVL_PREK[p % _NLVL] for p in range(_NP)]
    kept_tiles = []
    for r in range(8):
        for p in range(_NP):
            crow = [c8[p][i][r:r + 1] for i in range(4)]
            cT = [_col_bcast(c) for c in crow]
            areaT = jnp.maximum(cT[2] - cT[0], 0.0) * \
                jnp.maximum(cT[3] - cT[1], 0.0)
            iou_scr[p] = _iou_strip(cT, areaT, crow)
            if r > 0:
                scol = None
                for rp in range(r):
                    cprev = [c8[p][i][rp:rp + 1] for i in range(4)]
                    strip = _iou_strip(cT, areaT, cprev)
                    kmask = kept_tiles[rp][p:p + 1] > 0.5
                    hit = jnp.max(
                        jnp.where((strip > NMS_IOU) & kmask, 1.0, 0.0),
                        axis=1, keepdims=True)
                    scol = hit if scol is None else jnp.maximum(scol, hit)
                srow = _col_to_row(scol)
            else:
                srow = jnp.zeros((1, 128), jnp.float32)
            vrow = jnp.where((lane + r * 128) < prek[p], 1.0, 0.0)
            arow = jnp.where(srow > 0.5, 0.0, vrow)
            if p == 0:
                act_rows = [arow]
            else:
                act_rows.append(arow)
        act = jnp.concatenate(act_rows, axis=0)

        for blk_i in range(16):
            blk = iou_scr[:, blk_i * 8:(blk_i + 1) * 8, :]
            for s in range(8):
                jj = blk_i * 8 + s
                iou_row = blk[:, s, :]
                aj = jnp.max(jnp.where(lane == jj, act, 0.0),
                             axis=1, keepdims=True)
                supp = (iou_row > NMS_IOU) & (lane > jj) & (aj > 0.5)
                act = jnp.where(supp, 0.0, act)
        kept_tiles.append(act)

    kept8 = [jnp.concatenate([kept_tiles[r][p:p + 1] for r in range(8)], axis=0)
             for p in range(_NP)]
    masked = jnp.concatenate(
        [jnp.where(kept8[p] > 0.5, k8[p], -1.0) for p in range(_NP)], axis=0)
    idx_all = jnp.concatenate(idx8, axis=0)
    coords_all = [jnp.concatenate([c8[p][i] for p in range(_NP)], axis=0)
                  for i in range(4)]
    mkey, _, mc = _bitonic_sort_desc(masked, idx_all, coords_all, 8)
    msc = jnp.maximum(mkey, 0.0)
    mc = [jnp.where(mkey > -0.5, c, 0.0) for c in mc]

    pad = jnp.full((8, 128), -1.0, dtype=jnp.float32)
    zpad = jnp.zeros((8, 128), dtype=jnp.float32)
    sc_chunks = []
    c_chunks = [[] for _ in range(4)]
    for b in range(_B):
        for l in range(_NLVL):
            p = b * _NLVL + l
            sc_chunks.append(msc[p * 8:(p + 1) * 8])
            for i in range(4):
                c_chunks[i].append(mc[i][p * 8:(p + 1) * 8])
        sc_chunks.append(pad)
        for i in range(4):
            c_chunks[i].append(zpad)
    allsc = jnp.concatenate(sc_chunks, axis=0)
    allc = [jnp.concatenate(ch, axis=0) for ch in c_chunks]
    row128 = lax.broadcasted_iota(jnp.int32, (128, 1), 0)
    mpos = (row128 & 31) * 128 + \
        lax.broadcasted_iota(jnp.int32, (128, 128), 1)
    fkey, _, fc = _bitonic_sort_desc(allsc, mpos, allc, 32)
    for b in range(_B):
        sc_out[b] = fkey[b * 32:b * 32 + 8]
        for i in range(4):
            bx_out[b, i] = fc[i][b * 32:b * 32 + 8]


def _prep_level(rs, rb, ab, n2):
    B = rs.shape[0]
    nb = rs.shape[1] * rs.shape[2] * rs.shape[3]
    R = n2 // 128
    sc = jnp.reshape(rs, (B, nb))
    sc = jnp.pad(sc, ((0, 0), (0, n2 - nb)), constant_values=-jnp.inf)
    sc = jnp.reshape(sc, (B, R, 128))
    bx = jnp.transpose(jnp.reshape(rb, (B, nb, 4)), (0, 2, 1))
    bx = jnp.reshape(jnp.pad(bx, ((0, 0), (0, 0), (0, n2 - nb))), (B, 4, R, 128))
    an = jnp.transpose(jnp.reshape(ab, (B, nb, 4)), (0, 2, 1))
    an = jnp.reshape(jnp.pad(an, ((0, 0), (0, 0), (0, n2 - nb))), (B, 4, R, 128))
    return sc, bx, an


@functools.partial(jax.jit, static_argnames=("interpret",))
def _run(raw_boxes_l3, raw_scores_l3, anchor_boxes_l3,
         raw_boxes_l4, raw_scores_l4, anchor_boxes_l4,
         raw_boxes_l5, raw_scores_l5, anchor_boxes_l5,
         image_shape, interpret=False):
    B = raw_scores_l3.shape[0]
    sc3, bx3, an3 = _prep_level(raw_scores_l3, raw_boxes_l3, anchor_boxes_l3, 16384)
    sc4, bx4, an4 = _prep_level(raw_scores_l4, raw_boxes_l4, anchor_boxes_l4, 4096)
    sc5, bx5, an5 = _prep_level(raw_scores_l5, raw_boxes_l5, anchor_boxes_l5, 1024)

    def bs(shape):
        return pl.BlockSpec(shape, lambda: (0,) * len(shape))

    sc_out, bx_out = pl.pallas_call(
        _roi_kernel,
        in_specs=[
            bs((B, 128, 128)), bs((B, 4, 128, 128)), bs((B, 4, 128, 128)),
            bs((B, 32, 128)), bs((B, 4, 32, 128)), bs((B, 4, 32, 128)),
            bs((B, 8, 128)), bs((B, 4, 8, 128)), bs((B, 4, 8, 128)),
            pl.BlockSpec(memory_space=pltpu.SMEM),
        ],
        out_specs=[bs((B, 8, 128)), bs((B, 4, 8, 128))],
        out_shape=[
            jax.ShapeDtypeStruct((B, 8, 128), jnp.float32),
            jax.ShapeDtypeStruct((B, 4, 8, 128), jnp.float32),
        ],
        scratch_shapes=[
            pltpu.VMEM((_NP, 128, 128), jnp.float32),
        ],
        interpret=interpret,
    )(sc3, bx3, an3, sc4, bx4, an4, sc5, bx5, an5, image_shape)

    scores = jnp.reshape(sc_out, (B, 1024))[:, :NUM_PROPOSALS]
    rois = jnp.transpose(jnp.reshape(bx_out, (B, 4, 1024)), (0, 2, 1))[:, :NUM_PROPOSALS, :]
    return rois, scores


def kernel(raw_boxes_l3, raw_scores_l3, anchor_boxes_l3,
           raw_boxes_l4, raw_scores_l4, anchor_boxes_l4,
           raw_boxes_l5, raw_scores_l5, anchor_boxes_l5,
           image_shape):
    return _run(raw_boxes_l3, raw_scores_l3, anchor_boxes_l3,
                raw_boxes_l4, raw_scores_l4, anchor_boxes_l4,
                raw_boxes_l5, raw_scores_l5, anchor_boxes_l5,
                image_shape)


# speedup vs baseline: 42.9993x; 1.0380x over previous
import functools

import jax
import jax.numpy as jnp
import numpy as np
from jax import lax
from jax.experimental import pallas as pl
from jax.experimental.pallas import tpu as pltpu

BBOX_XFORM_CLIP = float(np.log(1000.0 / 16.0))
PRE_NMS_TOP_K = 1000
NUM_PROPOSALS = 1000
NMS_IOU = 0.7

_B = 4
_NLVL = 3
_NP = _B * _NLVL
_LVL_N2 = (16384, 4096, 1024)
_LVL_NB = (12288, 3072, 768)
_LVL_PREK = tuple(min(nb, PRE_NMS_TOP_K) for nb in _LVL_NB)


def _lane_iota(R=1):
    return lax.broadcasted_iota(jnp.int32, (R, 128), 1)[0:1]


def _cmp_pass(key, idx, vals, j, m_lo, d):
    R = key.shape[0]
    if j < 128:
        axis, jr, sz = 1, j, 128
    else:
        axis, jr, sz = 0, j // 128, R

    def prt(v):
        return jnp.where(m_lo, pltpu.roll(v, sz - jr, axis),
                         pltpu.roll(v, jr, axis))

    pk = prt(key)
    pidx = prt(idx)
    pvals = [prt(v) for v in vals]
    keep_max = m_lo == d
    eq = pk == key
    gt = (pk > key) | (eq & (pidx < idx))
    lt = (pk < key) | (eq & (pidx > idx))
    take = (keep_max & gt) | (~keep_max & lt)
    key = jnp.where(take, pk, key)
    idx = jnp.where(take, pidx, idx)
    vals = [jnp.where(take, pv, v) for pv, v in zip(pvals, vals)]
    return key, idx, vals


def _sort_masks(R, R_local, j, k):
    lane = lax.broadcasted_iota(jnp.int32, (R, 128), 1)[0:1]
    row = lax.broadcasted_iota(jnp.int32, (R, 1), 0)
    rowl = row & (R_local - 1)
    if j < 128:
        m_lo = (lane & j) == 0
    else:
        m_lo = (rowl & (j // 128)) == 0
    if k < 128:
        d = (lane & k) == 0
    else:
        d = (rowl & (k // 128)) == 0
    return m_lo, d


def _bitonic_sort_desc(key, idx, vals, R_local, alt_dir=False):
    R = key.shape[0]
    n = R_local * 128
    row = lax.broadcasted_iota(jnp.int32, (R, 1), 0)
    bpar = ((row >> 3) & 1) == 1
    k = 2
    while k <= n:
        j = k // 2
        while j >= 1:
            m_lo, d = _sort_masks(R, R_local, j, k)
            if alt_dir:
                d = d ^ bpar
            key, idx, vals = _cmp_pass(key, idx, vals, j, m_lo, d)
            j //= 2
        k *= 2
    return key, idx, vals


def _topk1024_desc(key, idx, vals, bpp):
    if bpp == 1:
        return _bitonic_sort_desc(key, idx, vals, 8)
    key, idx, vals = _bitonic_sort_desc(key, idx, vals, 8, alt_dir=True)
    while bpp > 1:
        R = key.shape[0]
        row = lax.broadcasted_iota(jnp.int32, (R, 1), 0)
        m_lo = ((row >> 3) & 1) == 0
        d = jnp.broadcast_to(jnp.bool_(True), (R, 1))
        key, idx, vals = _cmp_pass(key, idx, vals, 1024, m_lo, d)
        def drop(v, R=R):
            return jnp.concatenate([v[g * 16:g * 16 + 8] for g in range(R // 16)],
                                   axis=0)
        key = drop(key)
        idx = drop(idx)
        vals = [drop(v) for v in vals]
        bpp //= 2
        R = key.shape[0]
        row = lax.broadcasted_iota(jnp.int32, (R, 1), 0)
        bpar = ((row >> 3) & 1) == 1
        j = 512
        while j >= 1:
            m_lo, _ = _sort_masks(R, 8, j, 1024)
            if bpp == 1:
                d = jnp.broadcast_to(jnp.bool_(True), (R, 1))
            else:
                d = ~bpar
            key, idx, vals = _cmp_pass(key, idx, vals, j, m_lo, d)
            j //= 2
    return key, idx, vals


def _col_bcast(rowvec):
    return jnp.transpose(jnp.broadcast_to(rowvec, (128, 128)), (1, 0))


def _col_to_row(col):
    return jnp.transpose(jnp.broadcast_to(col, (128, 128)), (1, 0))[0:1]


def _decode_clip(bx, an, hcol, wcol):
    ay1, ax1, ay2, ax2 = an
    dy, dx, dh, dw = bx
    ah = ay2 - ay1
    aw = ax2 - ax1
    ayc = ay1 + 0.5 * ah
    axc = ax1 + 0.5 * aw
    dh = jnp.minimum(dh, BBOX_XFORM_CLIP)
    dw = jnp.minimum(dw, BBOX_XFORM_CLIP)
    nyc = dy * ah + ayc
    nxc = dx * aw + axc
    nh = jnp.exp(dh) * ah
    nw = jnp.exp(dw) * aw
    zero = jnp.float32(0.0)
    y1 = jnp.clip(nyc - 0.5 * nh, zero, hcol)
    x1 = jnp.clip(nxc - 0.5 * nw, zero, wcol)
    y2 = jnp.clip(nyc + 0.5 * nh, zero, hcol)
    x2 = jnp.clip(nxc + 0.5 * nw, zero, wcol)
    return [y1, x1, y2, x2]


def _iou_strip(cT, areaT, crow):
    y1T, x1T, y2T, x2T = cT
    y1B = jnp.broadcast_to(crow[0], (128, 128))
    x1B = jnp.broadcast_to(crow[1], (128, 128))
    y2B = jnp.broadcast_to(crow[2], (128, 128))
    x2B = jnp.broadcast_to(crow[3], (128, 128))
    areaB = jnp.maximum(y2B - y1B, 0.0) * jnp.maximum(x2B - x1B, 0.0)
    iy = jnp.maximum(jnp.minimum(y2T, y2B) - jnp.maximum(y1T, y1B), 0.0)
    ix = jnp.maximum(jnp.minimum(x2T, x2B) - jnp.maximum(x1T, x1B), 0.0)
    inter = iy * ix
    return inter / (areaT + areaB - inter + 1e-8)


def _roi_kernel(sc3, bx3, an3, sc4, bx4, an4, sc5, bx5, an5, img,
                sc_out, bx_out, iou_scr):
    lane = _lane_iota()

    k8 = [None] * _NP
    idx8 = [None] * _NP
    c8 = [None] * _NP
    for l, (sc_ref, bx_ref, an_ref) in enumerate(
            ((sc3, bx3, an3), (sc4, bx4, an4), (sc5, bx5, an5))):
        R = _LVL_N2[l] // 128
        key = jax.nn.sigmoid(
            jnp.concatenate([sc_ref[b] for b in range(_B)], axis=0))
        RT = _B * R
        row = lax.broadcasted_iota(jnp.int32, (RT, 1), 0)
        pos = (row & (R - 1)) * 128 + \
            lax.broadcasted_iota(jnp.int32, (RT, 128), 1)
        coords_raw = [jnp.concatenate([bx_ref[b, i] for b in range(_B)], axis=0)
                      for i in range(4)]
        anchors = [jnp.concatenate([an_ref[b, i] for b in range(_B)], axis=0)
                   for i in range(4)]
        hcol = jnp.concatenate(
            [jnp.zeros((R, 1), jnp.float32) + img[b, 0] for b in range(_B)], axis=0)
        wcol = jnp.concatenate(
            [jnp.zeros((R, 1), jnp.float32) + img[b, 1] for b in range(_B)], axis=0)
        coords = _decode_clip(coords_raw, anchors, hcol, wcol)
        key_s, idx_s, coords_s = _topk1024_desc(key, pos, coords, R // 8)
        for b in range(_B):
            p = b * _NLVL + l
            k8[p] = key_s[b * 8:b * 8 + 8]
            idx8[p] = idx_s[b * 8:b * 8 + 8]
            c8[p] = [c[b * 8:b * 8 + 8] for c in coords_s]

    prek = [_LVL_PREK[p % _NLVL] for p in range(_NP)]
    kept_tiles = []
    for r in range(8):
        for p in range(_NP):
            crow = [c8[p][i][r:r + 1] for i in range(4)]
            cT = [_col_bcast(c) for c in crow]
            areaT = jnp.maximum(cT[2] - cT[0], 0.0) * \
                jnp.maximum(cT[3] - cT[1], 0.0)
            iou_scr[p] = _iou_strip(cT, areaT, crow)
            if r > 0:
                scol = None
                for rp in range(r):
                    cprev = [c8[p][i][rp:rp + 1] for i in range(4)]
                    strip = _iou_strip(cT, areaT, cprev)
                    kmask = kept_tiles[rp][p:p + 1] > 0.5
                    hit = jnp.max(
                        jnp.where((strip > NMS_IOU) & kmask, 1.0, 0.0),
                        axis=1, keepdims=True)
                    scol = hit if scol is None else jnp.maximum(scol, hit)
                srow = _col_to_row(scol)
            else:
                srow = jnp.zeros((1, 128), jnp.float32)
            vrow = jnp.where((lane + r * 128) < prek[p], 1.0, 0.0)
            arow = jnp.where(srow > 0.5, 0.0, vrow)
            if p == 0:
                act_rows = [arow]
            else:
                act_rows.append(arow)
        act = jnp.concatenate(act_rows, axis=0)

        for blk_i in range(16):
            blk = iou_scr[:, blk_i * 8:(blk_i + 1) * 8, :]
            for s in range(8):
                jj = blk_i * 8 + s
                iou_row = blk[:, s, :]
                aj = jnp.max(jnp.where(lane == jj, act, 0.0),
                             axis=1, keepdims=True)
                supp = (iou_row > NMS_IOU) & (lane > jj) & (aj > 0.5)
                act = jnp.where(supp, 0.0, act)
        kept_tiles.append(act)

    kept8 = [jnp.concatenate([kept_tiles[r][p:p + 1] for r in range(8)], axis=0)
             for p in range(_NP)]
    masked = jnp.concatenate(
        [jnp.where(kept8[p] > 0.5, k8[p], -1.0) for p in range(_NP)], axis=0)
    idx_all = jnp.concatenate(idx8, axis=0)
    coords_all = [jnp.concatenate([c8[p][i] for p in range(_NP)], axis=0)
                  for i in range(4)]
    mkey, _, mc = _bitonic_sort_desc(masked, idx_all, coords_all, 8)
    msc = jnp.maximum(mkey, 0.0)
    mc = [jnp.where(mkey > -0.5, c, 0.0) for c in mc]

    pad = jnp.full((8, 128), -1.0, dtype=jnp.float32)
    zpad = jnp.zeros((8, 128), dtype=jnp.float32)
    sc_chunks = []
    c_chunks = [[] for _ in range(4)]
    for b in range(_B):
        for l in range(_NLVL):
            p = b * _NLVL + l
            sc_chunks.append(msc[p * 8:(p + 1) * 8])
            for i in range(4):
                c_chunks[i].append(mc[i][p * 8:(p + 1) * 8])
        sc_chunks.append(pad)
        for i in range(4):
            c_chunks[i].append(zpad)
    allsc = jnp.concatenate(sc_chunks, axis=0)
    allc = [jnp.concatenate(ch, axis=0) for ch in c_chunks]
    row128 = lax.broadcasted_iota(jnp.int32, (128, 1), 0)
    mpos = (row128 & 31) * 128 + \
        lax.broadcasted_iota(jnp.int32, (128, 128), 1)
    fkey, _, fc = _bitonic_sort_desc(allsc, mpos, allc, 32)
    for b in range(_B):
        sc_out[b] = fkey[b * 32:b * 32 + 8]
        for i in range(4):
            bx_out[b, i] = fc[i][b * 32:b * 32 + 8]


def _prep_level(rs, rb, ab, n2):
    B = rs.shape[0]
    nb = rs.shape[1] * rs.shape[2] * rs.shape[3]
    R = n2 // 128
    sc = jnp.reshape(rs, (B, nb))
    sc = jnp.pad(sc, ((0, 0), (0, n2 - nb)), constant_values=-jnp.inf)
    sc = jnp.reshape(sc, (B, R, 128))
    bx = jnp.transpose(jnp.reshape(rb, (B, nb, 4)), (0, 2, 1))
    bx = jnp.reshape(jnp.pad(bx, ((0, 0), (0, 0), (0, n2 - nb))), (B, 4, R, 128))
    an = jnp.transpose(jnp.reshape(ab, (B, nb, 4)), (0, 2, 1))
    an = jnp.reshape(jnp.pad(an, ((0, 0), (0, 0), (0, n2 - nb))), (B, 4, R, 128))
    return sc, bx, an


@functools.partial(jax.jit, static_argnames=("interpret",))
def _run(raw_boxes_l3, raw_scores_l3, anchor_boxes_l3,
         raw_boxes_l4, raw_scores_l4, anchor_boxes_l4,
         raw_boxes_l5, raw_scores_l5, anchor_boxes_l5,
         image_shape, interpret=False):
    B = raw_scores_l3.shape[0]
    sc3, bx3, an3 = _prep_level(raw_scores_l3, raw_boxes_l3, anchor_boxes_l3, 16384)
    sc4, bx4, an4 = _prep_level(raw_scores_l4, raw_boxes_l4, anchor_boxes_l4, 4096)
    sc5, bx5, an5 = _prep_level(raw_scores_l5, raw_boxes_l5, anchor_boxes_l5, 1024)

    def bs(shape):
        return pl.BlockSpec(shape, lambda: (0,) * len(shape))

    sc_out, bx_out = pl.pallas_call(
        _roi_kernel,
        in_specs=[
            bs((B, 128, 128)), bs((B, 4, 128, 128)), bs((B, 4, 128, 128)),
            bs((B, 32, 128)), bs((B, 4, 32, 128)), bs((B, 4, 32, 128)),
            bs((B, 8, 128)), bs((B, 4, 8, 128)), bs((B, 4, 8, 128)),
            pl.BlockSpec(memory_space=pltpu.SMEM),
        ],
        out_specs=[bs((B, 8, 128)), bs((B, 4, 8, 128))],
        out_shape=[
            jax.ShapeDtypeStruct((B, 8, 128), jnp.float32),
            jax.ShapeDtypeStruct((B, 4, 8, 128), jnp.float32),
        ],
        scratch_shapes=[
            pltpu.VMEM((_NP, 128, 128), jnp.float32),
        ],
        interpret=interpret,
    )(sc3, bx3, an3, sc4, bx4, an4, sc5, bx5, an5, image_shape)

    scores = jnp.reshape(sc_out, (B, 1024))[:, :NUM_PROPOSALS]
    rois = jnp.transpose(jnp.reshape(bx_out, (B, 4, 1024)), (0, 2, 1))[:, :NUM_PROPOSALS, :]
    return rois, scores


def kernel(raw_boxes_l3, raw_scores_l3, anchor_boxes_l3,
           raw_boxes_l4, raw_scores_l4, anchor_boxes_l4,
           raw_boxes_l5, raw_scores_l5, anchor_boxes_l5,
           image_shape):
    return _run(raw_boxes_l3, raw_scores_l3, anchor_boxes_l3,
                raw_boxes_l4, raw_scores_l4, anchor_boxes_l4,
                raw_boxes_l5, raw_scores_l5, anchor_boxes_l5,
                image_shape)
